# baseline (speedup 1.0000x reference)
import jax
import jax.numpy as jnp
from jax import lax
from jax.experimental import pallas as pl
from jax.experimental.pallas import tpu as pltpu
from jax.experimental.pallas import tpu_sc as plsc

R, E, DIN, DOUT = 10000, 320000, 128, 128
NC, NS = 2, 16
NW = NC * NS
EPW = E // NW
CHUNK = 80
NCHUNK = EPW // CHUNK
NBUF = 4
RPT = 624
RREM = R - NS * RPT
ZROWS = 48


def _sc_scatter_body(c_hbm, et_hbm, out_hbm,
                     idx_bufs, rows_bufs, sems, zbuf, acc_sh):
    cid = lax.axis_index("c")
    sid = lax.axis_index("s")
    wid = sid * NC + cid

    def rows_src(jj):
        return c_hbm.at[pl.ds(wid * EPW + jj * CHUNK, CHUNK)]

    def idx_src(jj):
        return et_hbm.at[pl.ds(wid * EPW + jj * CHUNK, CHUNK)]

    def start_load(jj, b):
        pltpu.async_copy(idx_src(jj), idx_bufs[b], sems[b])
        pltpu.async_copy(rows_src(jj), rows_bufs[b], sems[b])

    def wait_load(jj, b):
        pltpu.make_async_copy(idx_src(jj), idx_bufs[b], sems[b]).wait()
        pltpu.make_async_copy(rows_src(jj), rows_bufs[b], sems[b]).wait()

    def scatter(b):
        pltpu.sync_copy(rows_bufs[b], acc_sh.at[idx_bufs[b]], add=True)

    for b in range(NBUF):
        start_load(b, b)

    zv = jnp.zeros((16,), jnp.float32)

    def zfill(r, carry):
        for kk in range(DIN // 16):
            zbuf[r, pl.ds(kk * 16, 16)] = zv
        return carry

    lax.fori_loop(0, ZROWS, zfill, 0)
    for k in range(RPT // ZROWS):
        pltpu.sync_copy(zbuf, acc_sh.at[pl.ds(sid * RPT + k * ZROWS, ZROWS)])

    @pl.when(sid == 0)
    def _():
        pltpu.sync_copy(zbuf.at[pl.ds(0, RREM)],
                        acc_sh.at[pl.ds(NS * RPT, RREM)])

    plsc.subcore_barrier()

    def body(i, carry):
        j = i * NBUF
        for b in range(NBUF):
            jj = j + b
            wait_load(jj, b)
            scatter(b)

            @pl.when(jj + NBUF < NCHUNK)
            def _():
                start_load(jj + NBUF, b)

        return carry

    lax.fori_loop(0, NCHUNK // NBUF, body, 0)

    wait_load(NCHUNK - 1, 0)
    scatter(0)

    plsc.subcore_barrier()
    pltpu.sync_copy(acc_sh.at[pl.ds(sid * RPT, RPT)],
                    out_hbm.at[cid, pl.ds(sid * RPT, RPT)])

    @pl.when(sid == 0)
    def _():
        pltpu.sync_copy(acc_sh.at[pl.ds(NS * RPT, RREM)],
                        out_hbm.at[cid, pl.ds(NS * RPT, RREM)])


def _sc_scatter(c_ijk, edge_type):
    mesh = plsc.VectorSubcoreMesh(core_axis_name="c", subcore_axis_name="s")
    f = pl.kernel(
        _sc_scatter_body,
        mesh=mesh,
        out_type=jax.ShapeDtypeStruct((NC, R, DIN), jnp.float32),
        scratch_types=[
            [pltpu.VMEM((CHUNK,), jnp.int32) for _ in range(NBUF)],
            [pltpu.VMEM((CHUNK, DIN), jnp.float32) for _ in range(NBUF)],
            [pltpu.SemaphoreType.DMA for _ in range(NBUF)],
            pltpu.VMEM((ZROWS, DIN), jnp.float32),
            pltpu.VMEM_SHARED((R, DIN), jnp.float32),
        ],
    )
    return f(c_ijk, edge_type)


BLK = 10000


def _tc_dense_body(gi_ref, p_ref, wrel_ref, w_ref, out_ref):
    g = p_ref[0] + p_ref[1]
    gp = lax.dot_general(gi_ref[...], wrel_ref[...],
                         dimension_numbers=(((1,), (1,)), ((), ())),
                         preferred_element_type=jnp.float32)
    gp = gp + jnp.dot(g, w_ref[...],
                      preferred_element_type=jnp.float32)
    nrm = jnp.sqrt(jnp.sum(gp * gp, axis=-1, keepdims=True))
    out_ref[...] = gp / jnp.maximum(nrm, 1e-12)


def kernel(g_initial, c_ijk, W, W_rel, edge_type):
    partial = _sc_scatter(c_ijk, edge_type)
    out = pl.pallas_call(
        _tc_dense_body,
        grid=(R // BLK,),
        in_specs=[
            pl.BlockSpec((BLK, DIN), lambda i: (i, 0)),
            pl.BlockSpec((NC, BLK, DIN), lambda i: (0, i, 0)),
            pl.BlockSpec((DOUT, DIN), lambda i: (0, 0)),
            pl.BlockSpec((DIN, DOUT), lambda i: (0, 0)),
        ],
        out_specs=pl.BlockSpec((BLK, DOUT), lambda i: (i, 0)),
        out_shape=jax.ShapeDtypeStruct((R, DOUT), jnp.float32),
    )(g_initial, partial, W_rel, W)
    return out


# speedup vs baseline: 1.0082x; 1.0082x over previous
import jax
import jax.numpy as jnp
from jax import lax
from jax.experimental import pallas as pl
from jax.experimental.pallas import tpu as pltpu
from jax.experimental.pallas import tpu_sc as plsc

R, E, DIN, DOUT = 10000, 320000, 128, 128
NC, NS = 2, 16
NW = NC * NS
EPW = E // NW
CHUNK = 80
NCHUNK = EPW // CHUNK
NBUF = 4
RPT = 624
RREM = R - NS * RPT
ZROWS = 48


def _sc_scatter_body(c_hbm, et_hbm, out_hbm,
                     idx_bufs, rows_bufs, sems, zbuf, acc_sh):
    cid = lax.axis_index("c")
    sid = lax.axis_index("s")
    wid = sid * NC + cid

    def rows_src(jj):
        return c_hbm.at[pl.ds(wid * EPW + jj * CHUNK, CHUNK)]

    def idx_src(jj):
        return et_hbm.at[pl.ds(wid * EPW + jj * CHUNK, CHUNK)]

    def start_load(jj, b):
        pltpu.async_copy(idx_src(jj), idx_bufs[b], sems[b])
        pltpu.async_copy(rows_src(jj), rows_bufs[b], sems[b])

    def wait_load(jj, b):
        pltpu.make_async_copy(idx_src(jj), idx_bufs[b], sems[b]).wait()
        pltpu.make_async_copy(rows_src(jj), rows_bufs[b], sems[b]).wait()

    def scatter(b):
        pltpu.sync_copy(rows_bufs[b], acc_sh.at[idx_bufs[b]], add=True)

    for b in range(NBUF):
        start_load(b, b)

    zv = jnp.zeros((16,), jnp.float32)

    def zfill(r, carry):
        for kk in range(DIN // 16):
            zbuf[r, pl.ds(kk * 16, 16)] = zv
        return carry

    lax.fori_loop(0, ZROWS, zfill, 0)
    for k in range(RPT // ZROWS):
        pltpu.sync_copy(zbuf, acc_sh.at[pl.ds(sid * RPT + k * ZROWS, ZROWS)])

    @pl.when(sid == 0)
    def _():
        pltpu.sync_copy(zbuf.at[pl.ds(0, RREM)],
                        acc_sh.at[pl.ds(NS * RPT, RREM)])

    plsc.subcore_barrier()

    def body(i, carry):
        j = i * NBUF
        for b in range(NBUF):
            jj = j + b
            wait_load(jj, b)
            scatter(b)

            @pl.when(jj + NBUF < NCHUNK)
            def _():
                start_load(jj + NBUF, b)

        return carry

    lax.fori_loop(0, NCHUNK // NBUF, body, 0)

    wait_load(NCHUNK - 1, 0)
    scatter(0)

    plsc.subcore_barrier()
    pltpu.sync_copy(acc_sh.at[pl.ds(sid * RPT, RPT)],
                    out_hbm.at[cid, pl.ds(sid * RPT, RPT)])

    @pl.when(sid == 0)
    def _():
        pltpu.sync_copy(acc_sh.at[pl.ds(NS * RPT, RREM)],
                        out_hbm.at[cid, pl.ds(NS * RPT, RREM)])


def _sc_scatter(c_ijk, edge_type):
    mesh = plsc.VectorSubcoreMesh(core_axis_name="c", subcore_axis_name="s")
    f = pl.kernel(
        _sc_scatter_body,
        mesh=mesh,
        out_type=jax.ShapeDtypeStruct((NC, R, DIN), jnp.float32),
        scratch_types=[
            [pltpu.VMEM((CHUNK,), jnp.int32) for _ in range(NBUF)],
            [pltpu.VMEM((CHUNK, DIN), jnp.float32) for _ in range(NBUF)],
            [pltpu.SemaphoreType.DMA for _ in range(NBUF)],
            pltpu.VMEM((ZROWS, DIN), jnp.float32),
            pltpu.VMEM_SHARED((R, DIN), jnp.float32),
        ],
    )
    return f(c_ijk, edge_type)


BLK = 5000


def _tc_rel_body(gi_ref, wrel_ref, h_ref):
    h_ref[...] = lax.dot_general(gi_ref[...], wrel_ref[...],
                                 dimension_numbers=(((1,), (1,)), ((), ())),
                                 preferred_element_type=jnp.float32)


def _tc_dense_body(h_ref, p_ref, w_ref, out_ref):
    g = p_ref[0] + p_ref[1]
    gp = h_ref[...] + jnp.dot(g, w_ref[...],
                              preferred_element_type=jnp.float32)
    nrm = jnp.sqrt(jnp.sum(gp * gp, axis=-1, keepdims=True))
    out_ref[...] = gp / jnp.maximum(nrm, 1e-12)


def kernel(g_initial, c_ijk, W, W_rel, edge_type):
    partial = _sc_scatter(c_ijk, edge_type)
    h = pl.pallas_call(
        _tc_rel_body,
        grid=(R // BLK,),
        in_specs=[
            pl.BlockSpec((BLK, DIN), lambda i: (i, 0)),
            pl.BlockSpec((DOUT, DIN), lambda i: (0, 0)),
        ],
        out_specs=pl.BlockSpec((BLK, DOUT), lambda i: (i, 0)),
        out_shape=jax.ShapeDtypeStruct((R, DOUT), jnp.float32),
    )(g_initial, W_rel)
    out = pl.pallas_call(
        _tc_dense_body,
        grid=(R // BLK,),
        in_specs=[
            pl.BlockSpec((BLK, DOUT), lambda i: (i, 0)),
            pl.BlockSpec((NC, BLK, DIN), lambda i: (0, i, 0)),
            pl.BlockSpec((DIN, DOUT), lambda i: (0, 0)),
        ],
        out_specs=pl.BlockSpec((BLK, DOUT), lambda i: (i, 0)),
        out_shape=jax.ShapeDtypeStruct((R, DOUT), jnp.float32),
    )(h, partial, W)
    return out


# speedup vs baseline: 1.0144x; 1.0061x over previous
import jax
import jax.numpy as jnp
from jax import lax
from jax.experimental import pallas as pl
from jax.experimental.pallas import tpu as pltpu
from jax.experimental.pallas import tpu_sc as plsc

R, E, DIN, DOUT = 10000, 320000, 128, 128
NC, NS = 2, 16
NW = NC * NS
EPW = E // NW
CHUNK = 80
NCHUNK = EPW // CHUNK
NBUF = 4
RPT = 624
RREM = R - NS * RPT
ZROWS = 48


def _sc_scatter_body(c_hbm, et_hbm, out_hbm,
                     idx_bufs, rows_bufs, sems, zbuf, acc_sh):
    cid = lax.axis_index("c")
    sid = lax.axis_index("s")
    wid = sid * NC + cid

    def rows_src(jj):
        return c_hbm.at[pl.ds(wid * EPW + jj * CHUNK, CHUNK)]

    def idx_src(jj):
        return et_hbm.at[pl.ds(wid * EPW + jj * CHUNK, CHUNK)]

    def start_load(jj, b):
        pltpu.async_copy(idx_src(jj), idx_bufs[b], sems[b])
        pltpu.async_copy(rows_src(jj), rows_bufs[b], sems[b])

    def wait_load(jj, b):
        pltpu.make_async_copy(idx_src(jj), idx_bufs[b], sems[b]).wait()
        pltpu.make_async_copy(rows_src(jj), rows_bufs[b], sems[b]).wait()

    def scatter(b):
        pltpu.sync_copy(rows_bufs[b], acc_sh.at[idx_bufs[b]], add=True)

    for b in range(NBUF):
        start_load(b, b)

    zv = jnp.zeros((16,), jnp.float32)

    def zfill(r, carry):
        for kk in range(DIN // 16):
            zbuf[r, pl.ds(kk * 16, 16)] = zv
        return carry

    lax.fori_loop(0, ZROWS, zfill, 0)
    for k in range(RPT // ZROWS):
        pltpu.sync_copy(zbuf, acc_sh.at[pl.ds(sid * RPT + k * ZROWS, ZROWS)])

    @pl.when(sid == 0)
    def _():
        pltpu.sync_copy(zbuf.at[pl.ds(0, RREM)],
                        acc_sh.at[pl.ds(NS * RPT, RREM)])

    plsc.subcore_barrier()

    def body(i, carry):
        j = i * NBUF
        for b in range(NBUF):
            jj = j + b
            wait_load(jj, b)
            scatter(b)

            @pl.when(jj + NBUF < NCHUNK)
            def _():
                start_load(jj + NBUF, b)

        return carry

    lax.fori_loop(0, NCHUNK // NBUF, body, 0)

    wait_load(NCHUNK - 1, 0)
    scatter(0)

    plsc.subcore_barrier()
    pltpu.sync_copy(acc_sh.at[pl.ds(sid * RPT, RPT)],
                    out_hbm.at[cid, pl.ds(sid * RPT, RPT)])

    @pl.when(sid == 0)
    def _():
        pltpu.sync_copy(acc_sh.at[pl.ds(NS * RPT, RREM)],
                        out_hbm.at[cid, pl.ds(NS * RPT, RREM)])


def _sc_scatter(c_ijk, edge_type):
    mesh = plsc.VectorSubcoreMesh(core_axis_name="c", subcore_axis_name="s")
    f = pl.kernel(
        _sc_scatter_body,
        mesh=mesh,
        out_type=jax.ShapeDtypeStruct((NC, R, DIN), jnp.float32),
        scratch_types=[
            [pltpu.VMEM((CHUNK,), jnp.int32) for _ in range(NBUF)],
            [pltpu.VMEM((CHUNK, DIN), jnp.float32) for _ in range(NBUF)],
            [pltpu.SemaphoreType.DMA for _ in range(NBUF)],
            pltpu.VMEM((ZROWS, DIN), jnp.float32),
            pltpu.VMEM_SHARED((R, DIN), jnp.float32),
        ],
    )
    return f(c_ijk, edge_type)


BLK = 5000


def _tc_dense_body(gi_ref, p_ref, wrel_ref, w_ref, out_ref):
    g = p_ref[0] + p_ref[1]
    gp = lax.dot_general(gi_ref[...], wrel_ref[...],
                         dimension_numbers=(((1,), (1,)), ((), ())),
                         preferred_element_type=jnp.float32)
    gp = gp + jnp.dot(g, w_ref[...],
                      preferred_element_type=jnp.float32)
    nrm = jnp.sqrt(jnp.sum(gp * gp, axis=-1, keepdims=True))
    out_ref[...] = gp / jnp.maximum(nrm, 1e-12)


def kernel(g_initial, c_ijk, W, W_rel, edge_type):
    partial = _sc_scatter(c_ijk, edge_type)
    out = pl.pallas_call(
        _tc_dense_body,
        grid=(R // BLK,),
        in_specs=[
            pl.BlockSpec((BLK, DIN), lambda i: (i, 0)),
            pl.BlockSpec((NC, BLK, DIN), lambda i: (0, i, 0)),
            pl.BlockSpec((DOUT, DIN), lambda i: (0, 0)),
            pl.BlockSpec((DIN, DOUT), lambda i: (0, 0)),
        ],
        out_specs=pl.BlockSpec((BLK, DOUT), lambda i: (i, 0)),
        out_shape=jax.ShapeDtypeStruct((R, DOUT), jnp.float32),
    )(g_initial, partial, W_rel, W)
    return out
